# Initial kernel scaffold; baseline (speedup 1.0000x reference)
import jax
import jax.numpy as jnp
from jax.experimental import pallas as pl


def kernel(x, knn, XNorm, YNorm, W1, b1, W2, b2, Wd1, bd1, Wd2, bd2):
    raise NotImplementedError("write your pallas kernel here")


# speedup vs baseline: 1.6828x; 1.6828x over previous
import functools

import jax
import jax.numpy as jnp
from jax import lax
from jax.experimental import pallas as pl
from jax.experimental.pallas import tpu as pltpu
from jax.experimental.pallas import tpu_sc as plsc

C = 64
D = 64
EPS = 1e-20
_HIGHEST = lax.Precision.HIGHEST


def _encoder_body(x_ref, xnorm_ref, w1_ref, b1_ref, w2_ref, b2_ref, out_ref):
    xn = (x_ref[...] - xnorm_ref[0:1, :]) / xnorm_ref[1:2, :]
    h = jnp.dot(xn, w1_ref[...],
                preferred_element_type=jnp.float32) + b1_ref[...][None, :]
    h = jnp.maximum(h, 0.0)
    out_ref[...] = jnp.dot(h, w2_ref[...],
                           preferred_element_type=jnp.float32) + b2_ref[...][None, :]


def _encoder(x, XNorm, W1, b1, W2, b2):
    Bsz, x_dim = x.shape
    hidden = W1.shape[1]
    latent = W2.shape[1]
    bt = 128
    grid = (Bsz // bt,)
    return pl.pallas_call(
        _encoder_body,
        grid=grid,
        in_specs=[
            pl.BlockSpec((bt, x_dim), lambda i: (i, 0)),
            pl.BlockSpec((2, x_dim), lambda i: (0, 0)),
            pl.BlockSpec((x_dim, hidden), lambda i: (0, 0)),
            pl.BlockSpec((hidden,), lambda i: (0,)),
            pl.BlockSpec((hidden, latent), lambda i: (0, 0)),
            pl.BlockSpec((latent,), lambda i: (0,)),
        ],
        out_specs=pl.BlockSpec((bt, latent), lambda i: (i, 0)),
        out_shape=jax.ShapeDtypeStruct((Bsz, latent), jnp.float32),
        compiler_params=pltpu.CompilerParams(
            dimension_semantics=("arbitrary",)),
    )(x, XNorm, W1, b1, W2, b2)


def _sample_body(knn_ref, logits_ref, u_ref, offs_ref):
    k = pl.program_id(0)
    scale = knn_ref[k]
    noise = u_ref[...] - 0.5
    samples = scale * noise + 0.5
    g = -jnp.log(-jnp.log(samples + EPS) + EPS)
    s = logits_ref[...] + g
    m = jnp.max(s, axis=-1, keepdims=True)
    li = lax.broadcasted_iota(jnp.int32, s.shape, 2)
    cand = jnp.where(s == m, li, D)
    idx = jnp.min(cand, axis=-1)
    ci = lax.broadcasted_iota(jnp.int32, idx.shape, 1)
    offs_ref[...] = ci * D + idx


def _sample(logits3, u3, knn):
    Bsz = logits3.shape[0]
    K = knn.shape[0]
    bt = 256
    grid = (K, Bsz // bt)
    return pl.pallas_call(
        _sample_body,
        grid=grid,
        in_specs=[
            pl.BlockSpec(memory_space=pltpu.SMEM),
            pl.BlockSpec((bt, C, D), lambda k, t: (t, 0, 0)),
            pl.BlockSpec((bt, C, D), lambda k, t: (k * (Bsz // bt) + t, 0, 0)),
        ],
        out_specs=pl.BlockSpec((bt, C), lambda k, t: (k * (Bsz // bt) + t, 0)),
        out_shape=jax.ShapeDtypeStruct((K * Bsz, C), jnp.int32),
        compiler_params=pltpu.CompilerParams(
            dimension_semantics=("arbitrary", "arbitrary")),
    )(knn, logits3, u3)


def _build_estimate_sc(offs):
    N = offs.shape[0]
    latent = C * D
    info = plsc.get_sparse_core_info()
    nw = info.num_cores * info.num_subcores
    rows_per_w = N // nw
    grp = 8
    n_grps = rows_per_w // grp
    mesh = plsc.VectorSubcoreMesh(core_axis_name="c", subcore_axis_name="s")
    offs_flat = offs.reshape(N * C)

    @functools.partial(
        pl.kernel, mesh=mesh,
        out_type=jax.ShapeDtypeStruct((N * latent,), jnp.float32),
        scratch_types=[
            pltpu.VMEM((grp * latent,), jnp.float32),
            pltpu.VMEM((grp * C,), jnp.int32),
        ],
        compiler_params=pltpu.CompilerParams(use_tc_tiling_on_sc=False,
                                             needs_layout_passes=False),
    )
    def est_kernel(offs_hbm, est_hbm, buf, offs_v):
        wid = lax.axis_index("s") * info.num_cores + lax.axis_index("c")
        base = wid * rows_per_w
        zeros16 = jnp.zeros((16,), jnp.float32)
        ones16 = jnp.ones((16,), jnp.float32)

        def _zbody(i, _):
            buf[pl.ds(i * 16, 16)] = zeros16
            return 0
        lax.fori_loop(0, grp * latent // 16, _zbody, 0)

        def _grp_body(g, _):
            rbase = base + g * grp
            pltpu.sync_copy(offs_hbm.at[pl.ds(rbase * C, grp * C)], offs_v)
            for r in range(grp):
                for j in range(C // 16):
                    ov = offs_v[pl.ds(r * C + j * 16, 16)] + (r * latent)
                    plsc.store_scatter(buf, [ov], ones16)
            pltpu.sync_copy(buf, est_hbm.at[pl.ds(rbase * latent, grp * latent)])
            for r in range(grp):
                for j in range(C // 16):
                    ov = offs_v[pl.ds(r * C + j * 16, 16)] + (r * latent)
                    plsc.store_scatter(buf, [ov], zeros16)
            return 0

        lax.fori_loop(0, n_grps, _grp_body, 0)

    return est_kernel(offs_flat).reshape(N, latent)


def _decoder_body(offs_ref, wd1_ref, bd1_ref, wd2_ref, bd2_ref, ynorm_ref,
                  y_ref):
    latent = wd1_ref.shape[0]
    offs = offs_ref[...].astype(jnp.float32)
    ecol = lax.broadcasted_iota(jnp.int32, (C, latent), 1) // D
    erow = lax.broadcasted_iota(jnp.int32, (C, latent), 0)
    E = (ecol == erow).astype(jnp.float32)
    expand = jnp.dot(offs, E, precision=_HIGHEST,
                     preferred_element_type=jnp.float32)
    lane = lax.broadcasted_iota(jnp.int32, expand.shape, 1)
    onehot = (expand.astype(jnp.int32) == lane).astype(jnp.bfloat16)
    hd = jnp.dot(onehot, wd1_ref[...],
                 preferred_element_type=jnp.float32) + bd1_ref[...][None, :]
    hd = jnp.maximum(hd, 0.0)
    yb = jnp.dot(hd.astype(jnp.bfloat16), wd2_ref[...],
                 preferred_element_type=jnp.float32) + bd2_ref[...][None, :]
    y_ref[...] = yb * ynorm_ref[1:2, :] + ynorm_ref[0:1, :]


def _decoder(offs, Wd1_bf, bd1, Wd2_bf, bd2, YNorm):
    N = offs.shape[0]
    latent, hidden = Wd1_bf.shape
    y_dim = Wd2_bf.shape[1]
    bt = 256
    grid = (N // bt,)
    return pl.pallas_call(
        _decoder_body,
        grid=grid,
        in_specs=[
            pl.BlockSpec((bt, C), lambda i: (i, 0)),
            pl.BlockSpec((latent, hidden), lambda i: (0, 0)),
            pl.BlockSpec((hidden,), lambda i: (0,)),
            pl.BlockSpec((hidden, y_dim), lambda i: (0, 0)),
            pl.BlockSpec((y_dim,), lambda i: (0,)),
            pl.BlockSpec((2, y_dim), lambda i: (0, 0)),
        ],
        out_specs=pl.BlockSpec((bt, y_dim), lambda i: (i, 0)),
        out_shape=jax.ShapeDtypeStruct((N, y_dim), jnp.float32),
        compiler_params=pltpu.CompilerParams(
            dimension_semantics=("arbitrary",)),
    )(offs, Wd1_bf, bd1, Wd2_bf, bd2, YNorm)


def kernel(x, knn, XNorm, YNorm, W1, b1, W2, b2, Wd1, bd1, Wd2, bd2):
    Bsz = x.shape[0]
    K = knn.shape[0]
    u = jax.random.uniform(jax.random.key(42), (K, Bsz, C, D),
                           dtype=jnp.float32)
    u3 = u.reshape(K * Bsz, C, D)

    logits = _encoder(x, XNorm, W1, b1, W2, b2)
    logits3 = logits.reshape(Bsz, C, D)
    offs = _sample(logits3, u3, knn)

    estimate = _build_estimate_sc(offs)
    y = _decoder(offs, Wd1.astype(jnp.bfloat16), bd1,
                 Wd2.astype(jnp.bfloat16), bd2, YNorm)
    return (y, estimate)
